# Initial kernel scaffold; baseline (speedup 1.0000x reference)
import functools

import jax
import jax.numpy as jnp
from jax import lax
from jax.experimental import pallas as pl
from jax.experimental.pallas import tpu as pltpu
from jax.experimental.pallas import tpu_sc as plsc

N_NODES = 10000
D = 128
E_EDGES = 320000

NC = 2
NS = 16
NW = NC * NS

K = 128
E_PAD = 327680
NCHUNK = E_PAD // (NW * K)
N_PAD = 10240
RPT = N_PAD // NS

DEG_R = 8

BLK = 2048
GRID_M = N_PAD // BLK

_mesh = plsc.VectorSubcoreMesh(core_axis_name="c", subcore_axis_name="s")


def _deg_body(ones_hbm, dsti_hbm, zeros_hbm, out_hbm, dst_idx, ones_v, dacc):
    c = lax.axis_index("c")
    s = lax.axis_index("s")
    wid = s * NC + c
    r0 = s * RPT
    pltpu.sync_copy(dsti_hbm.at[wid], dst_idx)
    pltpu.sync_copy(ones_hbm, ones_v)
    pltpu.sync_copy(zeros_hbm.at[pl.ds(r0, RPT)], dacc.at[pl.ds(r0, RPT)])
    plsc.subcore_barrier()

    def body(j, carry):
        pltpu.sync_copy(ones_v, dacc.at[dst_idx.at[j]], add=True)
        return carry

    lax.fori_loop(0, NCHUNK, body, 0)
    plsc.subcore_barrier()
    pltpu.sync_copy(dacc.at[pl.ds(r0, RPT)], out_hbm.at[c, pl.ds(r0, RPT)])


_deg_call = functools.partial(
    pl.kernel,
    _deg_body,
    out_type=jax.ShapeDtypeStruct((NC, N_PAD, DEG_R), jnp.float32),
    mesh=_mesh,
    scratch_types=[
        pltpu.VMEM((NCHUNK, K), jnp.int32),
        pltpu.VMEM((K, DEG_R), jnp.float32),
        pltpu.VMEM_SHARED((N_PAD, DEG_R), jnp.float32),
    ],
)()


def _agg_body(g_hbm, srci_hbm, dsti_hbm, zeros_hbm, out_hbm,
              src_idx, dst_idx, rows0, rows1, acc, sem0, sem1):
    c = lax.axis_index("c")
    s = lax.axis_index("s")
    wid = s * NC + c
    r0 = s * RPT
    pltpu.sync_copy(srci_hbm.at[wid], src_idx)
    pltpu.sync_copy(dsti_hbm.at[wid], dst_idx)

    @pl.when(c == 0)
    def _():
        pltpu.sync_copy(g_hbm.at[pl.ds(r0, RPT)], acc.at[pl.ds(r0, RPT)])

    @pl.when(c == 1)
    def _():
        pltpu.sync_copy(zeros_hbm.at[pl.ds(r0, RPT)], acc.at[pl.ds(r0, RPT)])

    plsc.subcore_barrier()

    rows = (rows0, rows1)
    sems = (sem0, sem1)

    def gather(j, b):
        return pltpu.make_async_copy(g_hbm.at[src_idx.at[j]], rows[b], sems[b])

    gather(0, 0).start()

    def step(j, b):
        @pl.when(j + 1 < NCHUNK)
        def _():
            gather(j + 1, 1 - b).start()

        gather(j, b).wait()
        pltpu.sync_copy(rows[b], acc.at[dst_idx.at[j]], add=True)

    def body(j2, carry):
        step(2 * j2, 0)
        step(2 * j2 + 1, 1)
        return carry

    lax.fori_loop(0, NCHUNK // 2, body, 0)
    plsc.subcore_barrier()
    pltpu.sync_copy(acc.at[pl.ds(r0, RPT)], out_hbm.at[c, pl.ds(r0, RPT)])


_agg_call = functools.partial(
    pl.kernel,
    _agg_body,
    out_type=jax.ShapeDtypeStruct((NC, N_PAD, D), jnp.float32),
    mesh=_mesh,
    scratch_types=[
        pltpu.VMEM((NCHUNK, K), jnp.int32),
        pltpu.VMEM((NCHUNK, K), jnp.int32),
        pltpu.VMEM((K, D), jnp.float32),
        pltpu.VMEM((K, D), jnp.float32),
        pltpu.VMEM_SHARED((N_PAD, D), jnp.float32),
        pltpu.SemaphoreType.DMA,
        pltpu.SemaphoreType.DMA,
    ],
)()


def _mm1_body(deg_ref, x_ref, w_ref, g_ref, dis_ref):
    deg = deg_ref[0] + deg_ref[1]
    dis8 = lax.rsqrt(deg + 1.0)
    dis_ref[...] = dis8
    dis1 = dis8[:, 0:1]
    g_ref[...] = dis1 * jnp.dot(x_ref[...], w_ref[...],
                                preferred_element_type=jnp.float32)


def _mm1_call(deg2, x_pad, w):
    return pl.pallas_call(
        _mm1_body,
        grid=(GRID_M,),
        in_specs=[
            pl.BlockSpec((NC, BLK, DEG_R), lambda i: (0, i, 0)),
            pl.BlockSpec((BLK, D), lambda i: (i, 0)),
            pl.BlockSpec((D, D), lambda i: (0, 0)),
        ],
        out_specs=[
            pl.BlockSpec((BLK, D), lambda i: (i, 0)),
            pl.BlockSpec((BLK, DEG_R), lambda i: (i, 0)),
        ],
        out_shape=[
            jax.ShapeDtypeStruct((N_PAD, D), jnp.float32),
            jax.ShapeDtypeStruct((N_PAD, DEG_R), jnp.float32),
        ],
    )(deg2, x_pad, w)


def _mid_body(acc_ref, dis_ref, b_ref, w_ref, g_ref):
    a = acc_ref[0] + acc_ref[1]
    dis1 = dis_ref[...][:, 0:1]
    h = jnp.maximum(dis1 * a + b_ref[...], 0.0)
    g_ref[...] = dis1 * jnp.dot(h, w_ref[...],
                                preferred_element_type=jnp.float32)


def _mid_call(acc, dis8, b, w):
    return pl.pallas_call(
        _mid_body,
        grid=(GRID_M,),
        in_specs=[
            pl.BlockSpec((NC, BLK, D), lambda i: (0, i, 0)),
            pl.BlockSpec((BLK, DEG_R), lambda i: (i, 0)),
            pl.BlockSpec((1, D), lambda i: (0, 0)),
            pl.BlockSpec((D, D), lambda i: (0, 0)),
        ],
        out_specs=pl.BlockSpec((BLK, D), lambda i: (i, 0)),
        out_shape=jax.ShapeDtypeStruct((N_PAD, D), jnp.float32),
    )(acc, dis8, b, w)


def _fin_body(acc_ref, dis_ref, b_ref, o_ref):
    a = acc_ref[0] + acc_ref[1]
    v = dis_ref[...][:, 0:1] * a + b_ref[...]
    m = jnp.max(v, axis=1, keepdims=True)
    z = v - m
    lse = jnp.log(jnp.sum(jnp.exp(z), axis=1, keepdims=True))
    o_ref[...] = z - lse


def _fin_call(acc, dis8, b):
    return pl.pallas_call(
        _fin_body,
        grid=(GRID_M,),
        in_specs=[
            pl.BlockSpec((NC, BLK, D), lambda i: (0, i, 0)),
            pl.BlockSpec((BLK, DEG_R), lambda i: (i, 0)),
            pl.BlockSpec((1, D), lambda i: (0, 0)),
        ],
        out_specs=pl.BlockSpec((BLK, D), lambda i: (i, 0)),
        out_shape=jax.ShapeDtypeStruct((N_PAD, D), jnp.float32),
    )(acc, dis8, b)


def kernel(x, edge_index, W1, b1, W2, b2, W3, b3):
    pad_e = E_PAD - E_EDGES
    pad_idx = jnp.full((pad_e,), N_NODES, jnp.int32)
    srcp = jnp.concatenate([edge_index[0], pad_idx]).reshape(NW, NCHUNK, K)
    dstp = jnp.concatenate([edge_index[1], pad_idx]).reshape(NW, NCHUNK, K)

    x_pad = jnp.pad(x, ((0, N_PAD - N_NODES), (0, 0)))
    zeros = jnp.zeros((N_PAD, D), jnp.float32)
    zeros8 = jnp.zeros((N_PAD, DEG_R), jnp.float32)
    ones8 = jnp.ones((K, DEG_R), jnp.float32)

    deg2 = _deg_call(ones8, dstp, zeros8)
    g, dis8 = _mm1_call(deg2, x_pad, W1)
    acc = _agg_call(g, srcp, dstp, zeros)
    g = _mid_call(acc, dis8, b1.reshape(1, D), W2)
    acc = _agg_call(g, srcp, dstp, zeros)
    g = _mid_call(acc, dis8, b2.reshape(1, D), W3)
    acc = _agg_call(g, srcp, dstp, zeros)
    out = _fin_call(acc, dis8, b3.reshape(1, D))
    return out[:N_NODES]


# speedup vs baseline: 9.0717x; 9.0717x over previous
import functools

import jax
import jax.numpy as jnp
from jax import lax
from jax.experimental import pallas as pl
from jax.experimental.pallas import tpu as pltpu
from jax.experimental.pallas import tpu_sc as plsc

N_NODES = 10000
D = 128
E_EDGES = 320000

NC = 2
NS = 16
NW = NC * NS

K = 128
E_PAD = 327680
NCHUNK = E_PAD // (NW * K)
NPHASE = 5
HC = NCHUNK // NPHASE
N_PAD = 10240
RPT = N_PAD // NS

DEG_R = 8

BLK = 2048
GRID_M = N_PAD // BLK


def _mesh():
    return plsc.VectorSubcoreMesh(core_axis_name="c", subcore_axis_name="s")


def _deg_body(ones_hbm, dsti_hbm, zeros_hbm, out_hbm, dst_idx, ones_v, dacc):
    c = lax.axis_index("c")
    s = lax.axis_index("s")
    wid = s * NC + c
    r0 = s * RPT
    pltpu.sync_copy(dsti_hbm.at[wid], dst_idx)
    pltpu.sync_copy(ones_hbm, ones_v)
    pltpu.sync_copy(zeros_hbm.at[pl.ds(r0, RPT)], dacc.at[pl.ds(r0, RPT)])
    plsc.subcore_barrier()

    def body(j, carry):
        pltpu.sync_copy(ones_v, dacc.at[dst_idx.at[j]], add=True)
        return carry

    lax.fori_loop(0, NCHUNK, body, 0)
    plsc.subcore_barrier()
    pltpu.sync_copy(dacc.at[pl.ds(r0, RPT)], out_hbm.at[c, pl.ds(r0, RPT)])


@functools.cache
def _deg_call():
    return pl.kernel(
        _deg_body,
        out_type=jax.ShapeDtypeStruct((NC, N_PAD, D), jnp.float32),
        mesh=_mesh(),
        scratch_types=[
            pltpu.VMEM((NCHUNK, K), jnp.int32),
            pltpu.VMEM((K, D), jnp.float32),
            pltpu.VMEM_SHARED((N_PAD, D), jnp.float32),
        ],
    )


def _agg_body(g_hbm, srci_hbm, dsti_hbm, zeros_hbm, out_hbm,
              src_idx, dst_idx, rows0, rows1, acc, sem0, sem1):
    c = lax.axis_index("c")
    s = lax.axis_index("s")
    wid = s * NC + c
    r0 = s * RPT

    @pl.when(c == 0)
    def _():
        pltpu.sync_copy(g_hbm.at[pl.ds(r0, RPT)], acc.at[pl.ds(r0, RPT)])

    @pl.when(c == 1)
    def _():
        pltpu.sync_copy(zeros_hbm.at[pl.ds(r0, RPT)], acc.at[pl.ds(r0, RPT)])

    plsc.subcore_barrier()

    rows = (rows0, rows1)
    sems = (sem0, sem1)

    def gather(j, b):
        return pltpu.make_async_copy(g_hbm.at[src_idx.at[j]], rows[b], sems[b])

    for phase in range(NPHASE):
        pltpu.sync_copy(srci_hbm.at[wid, pl.ds(phase * HC, HC)], src_idx)
        pltpu.sync_copy(dsti_hbm.at[wid, pl.ds(phase * HC, HC)], dst_idx)
        gather(0, 0).start()

        def step(j, b):
            @pl.when(j + 1 < HC)
            def _():
                gather(j + 1, 1 - b).start()

            gather(j, b).wait()
            pltpu.sync_copy(rows[b], acc.at[dst_idx.at[j]], add=True)

        def body(j2, carry):
            step(2 * j2, 0)
            step(2 * j2 + 1, 1)
            return carry

        lax.fori_loop(0, HC // 2, body, 0)

    plsc.subcore_barrier()
    pltpu.sync_copy(acc.at[pl.ds(r0, RPT)], out_hbm.at[c, pl.ds(r0, RPT)])


@functools.cache
def _agg_call():
    return pl.kernel(
        _agg_body,
        out_type=jax.ShapeDtypeStruct((NC, N_PAD, D), jnp.float32),
        mesh=_mesh(),
        scratch_types=[
            pltpu.VMEM((HC, K), jnp.int32),
            pltpu.VMEM((HC, K), jnp.int32),
            pltpu.VMEM((K, D), jnp.float32),
            pltpu.VMEM((K, D), jnp.float32),
            pltpu.VMEM_SHARED((N_PAD, D), jnp.float32),
            pltpu.SemaphoreType.DMA,
            pltpu.SemaphoreType.DMA,
        ],
    )


def _mm1_body(deg_ref, x_ref, w_ref, g_ref, dis_ref):
    deg = deg_ref[0][:, :DEG_R] + deg_ref[1][:, :DEG_R]
    dis8 = lax.rsqrt(deg + 1.0)
    dis_ref[...] = dis8
    dis1 = dis8[:, 0:1]
    g_ref[...] = dis1 * jnp.dot(x_ref[...], w_ref[...],
                                preferred_element_type=jnp.float32)


def _mm1_call(deg2, x_pad, w):
    return pl.pallas_call(
        _mm1_body,
        grid=(GRID_M,),
        in_specs=[
            pl.BlockSpec((NC, BLK, D), lambda i: (0, i, 0)),
            pl.BlockSpec((BLK, D), lambda i: (i, 0)),
            pl.BlockSpec((D, D), lambda i: (0, 0)),
        ],
        out_specs=[
            pl.BlockSpec((BLK, D), lambda i: (i, 0)),
            pl.BlockSpec((BLK, DEG_R), lambda i: (i, 0)),
        ],
        out_shape=[
            jax.ShapeDtypeStruct((N_PAD, D), jnp.float32),
            jax.ShapeDtypeStruct((N_PAD, DEG_R), jnp.float32),
        ],
    )(deg2, x_pad, w)


def _mid_body(acc_ref, dis_ref, b_ref, w_ref, g_ref):
    a = acc_ref[0] + acc_ref[1]
    dis1 = dis_ref[...][:, 0:1]
    h = jnp.maximum(dis1 * a + b_ref[...], 0.0)
    g_ref[...] = dis1 * jnp.dot(h, w_ref[...],
                                preferred_element_type=jnp.float32)


def _mid_call(acc, dis8, b, w):
    return pl.pallas_call(
        _mid_body,
        grid=(GRID_M,),
        in_specs=[
            pl.BlockSpec((NC, BLK, D), lambda i: (0, i, 0)),
            pl.BlockSpec((BLK, DEG_R), lambda i: (i, 0)),
            pl.BlockSpec((1, D), lambda i: (0, 0)),
            pl.BlockSpec((D, D), lambda i: (0, 0)),
        ],
        out_specs=pl.BlockSpec((BLK, D), lambda i: (i, 0)),
        out_shape=jax.ShapeDtypeStruct((N_PAD, D), jnp.float32),
    )(acc, dis8, b, w)


def _fin_body(acc_ref, dis_ref, b_ref, o_ref):
    a = acc_ref[0] + acc_ref[1]
    v = dis_ref[...][:, 0:1] * a + b_ref[...]
    m = jnp.max(v, axis=1, keepdims=True)
    z = v - m
    lse = jnp.log(jnp.sum(jnp.exp(z), axis=1, keepdims=True))
    o_ref[...] = z - lse


def _fin_call(acc, dis8, b):
    return pl.pallas_call(
        _fin_body,
        grid=(GRID_M,),
        in_specs=[
            pl.BlockSpec((NC, BLK, D), lambda i: (0, i, 0)),
            pl.BlockSpec((BLK, DEG_R), lambda i: (i, 0)),
            pl.BlockSpec((1, D), lambda i: (0, 0)),
        ],
        out_specs=pl.BlockSpec((BLK, D), lambda i: (i, 0)),
        out_shape=jax.ShapeDtypeStruct((N_PAD, D), jnp.float32),
    )(acc, dis8, b)


def kernel(x, edge_index, W1, b1, W2, b2, W3, b3):
    pad_e = E_PAD - E_EDGES
    pad_idx = jnp.full((pad_e,), N_NODES, jnp.int32)
    srcp = jnp.concatenate([edge_index[0], pad_idx]).reshape(NW, NCHUNK, K)
    dstp = jnp.concatenate([edge_index[1], pad_idx]).reshape(NW, NCHUNK, K)

    x_pad = jnp.pad(x, ((0, N_PAD - N_NODES), (0, 0)))
    zeros = jnp.zeros((N_PAD, D), jnp.float32)
    ones_k = jnp.ones((K, D), jnp.float32)

    deg2 = _deg_call()(ones_k, dstp, zeros)
    g, dis8 = _mm1_call(deg2, x_pad, W1)
    agg = _agg_call()
    acc = agg(g, srcp, dstp, zeros)
    g = _mid_call(acc, dis8, b1.reshape(1, D), W2)
    acc = agg(g, srcp, dstp, zeros)
    g = _mid_call(acc, dis8, b2.reshape(1, D), W3)
    acc = agg(g, srcp, dstp, zeros)
    out = _fin_call(acc, dis8, b3.reshape(1, D))
    return out[:N_NODES]
